# baseline (speedup 1.0000x reference)
import functools

import numpy as np

import jax
import jax.numpy as jnp
from jax import lax
from jax.experimental import pallas as pl
from jax.experimental.pallas import tpu as pltpu
from jax.experimental.pallas import tpu_sc as plsc

VOCAB = 1000
DIM = 64
SENT = 50
B = 4096
OUT = SENT + 1
WPR = DIM // 2

try:
    _info = plsc.get_sparse_core_info()
    _NC, _NS, _L = _info.num_cores, _info.num_subcores, _info.num_lanes
except Exception:
    _NC, _NS, _L = 2, 16, 16

NW = _NC * _NS
BPW = B // NW

_mesh = plsc.VectorSubcoreMesh(
    core_axis_name="c", subcore_axis_name="s",
    num_cores=_NC, num_subcores=_NS,
)

_TGROUPS = [(off, min(_L, SENT - off)) for off in range(0, SENT, _L)]


@functools.partial(
    pl.kernel,
    out_type=jax.ShapeDtypeStruct((B * DIM,), jnp.float32),
    mesh=_mesh,
    scratch_types=[
        pltpu.VMEM((VOCAB * WPR,), jnp.int32),
        pltpu.VMEM((BPW * SENT + _L,), jnp.int32),
        pltpu.VMEM((BPW * DIM,), jnp.float32),
        pltpu.SemaphoreType.DMA,
    ],
    compiler_params=pltpu.CompilerParams(needs_layout_passes=False),
)
def _sc_pool(emb_hbm, x_hbm, out_hbm, table_v, idx_v, pool_v, sem):
    w = lax.axis_index("s") * _NC + lax.axis_index("c")
    table_cp = pltpu.async_copy(emb_hbm, table_v, sem)
    pltpu.sync_copy(x_hbm.at[pl.ds(w * (BPW * SENT), BPW * SENT)],
                    idx_v.at[pl.ds(0, BPW * SENT)])
    table_cp.wait()

    def body(b, carry):
        bt = b * SENT
        accs = [jnp.zeros((_L,), jnp.float32) for _ in range(4)]
        for off, nj in _TGROUPS:
            toks = idx_v[pl.ds(bt + off, _L)]
            for j in range(nj):
                base = toks[j] * WPR
                for k in range(2):
                    v = table_v[pl.ds(base + k * _L, _L)]
                    lo = lax.bitcast_convert_type(
                        lax.shift_left(v, 16), jnp.float32)
                    hi = lax.bitcast_convert_type(v, jnp.float32)
                    accs[2 * k] = accs[2 * k] + lo
                    accs[2 * k + 1] = accs[2 * k + 1] + hi
        for k in range(4):
            pool_v[pl.ds(b * DIM + k * _L, _L)] = accs[k]
        return carry

    lax.fori_loop(0, BPW, body, jnp.int32(0))
    pltpu.sync_copy(pool_v, out_hbm.at[pl.ds(w * (BPW * DIM), BPW * DIM)])


def _head_body(p_ref, wt_ref, b_ref, o_ref):
    logits = jnp.dot(p_ref[...], wt_ref[...],
                     preferred_element_type=jnp.float32)
    logits = logits + b_ref[...]
    m = jnp.max(logits, axis=-1, keepdims=True)
    e = jnp.exp(logits - m)
    o_ref[...] = e * (1.0 / jnp.sum(e, axis=-1, keepdims=True))


_HEAD_BLOCK = 512
_head = pl.pallas_call(
    _head_body,
    grid=(B // _HEAD_BLOCK,),
    in_specs=[
        pl.BlockSpec((_HEAD_BLOCK, DIM), lambda i: (i, 0)),
        pl.BlockSpec((DIM, OUT), lambda i: (0, 0)),
        pl.BlockSpec((1, OUT), lambda i: (0, 0)),
    ],
    out_specs=pl.BlockSpec((_HEAD_BLOCK, OUT), lambda i: (i, 0)),
    out_shape=jax.ShapeDtypeStruct((B, OUT), jnp.float32),
)

_PERM = np.array(
    [32 * (p // 32) + 2 * (p % 16) + ((p // 16) % 2) for p in range(DIM)],
    dtype=np.int32,
)


def kernel(x, emb, W, b):
    emb_pk = lax.bitcast_convert_type(
        emb.astype(jnp.bfloat16).reshape(VOCAB * WPR, 2), jnp.int32)
    pooled_sum = _sc_pool(emb_pk, x.astype(jnp.int32).reshape(-1))
    pooled_sum = pooled_sum.reshape(B, DIM)
    wt = W.T.astype(jnp.float32)[_PERM] * (1.0 / SENT)
    return _head(pooled_sum, wt, b.reshape(1, OUT))


# speedup vs baseline: 1.5223x; 1.5223x over previous
import functools

import numpy as np

import jax
import jax.numpy as jnp
from jax import lax
from jax.experimental import pallas as pl
from jax.experimental.pallas import tpu as pltpu
from jax.experimental.pallas import tpu_sc as plsc

VOCAB = 1000
DIM = 64
SENT = 50
B = 4096
OUT = SENT + 1

try:
    _info = plsc.get_sparse_core_info()
    _NC, _NS, _L = _info.num_cores, _info.num_subcores, _info.num_lanes
except Exception:
    _NC, _NS, _L = 2, 16, 16

NW = _NC * _NS
BPW = B // NW

_mesh = plsc.VectorSubcoreMesh(
    core_axis_name="c", subcore_axis_name="s",
    num_cores=_NC, num_subcores=_NS,
)

_TGROUPS = [(0, range(_L)), (_L, range(_L)), (2 * _L, range(_L)),
            (SENT - _L, range(3 * _L - (SENT - _L), _L))]


@functools.partial(
    pl.kernel,
    out_type=jax.ShapeDtypeStruct((B, DIM), jnp.float32),
    mesh=_mesh,
    scratch_types=[
        pltpu.VMEM((VOCAB, DIM), jnp.bfloat16),
        pltpu.VMEM((BPW, SENT), jnp.int32),
        pltpu.VMEM((BPW, DIM), jnp.float32),
        pltpu.SemaphoreType.DMA,
    ],
    compiler_params=pltpu.CompilerParams(needs_layout_passes=False),
)
def _sc_pool(emb_hbm, x_hbm, out_hbm, table_v, idx_v, pool_v, sem):
    w = lax.axis_index("s") * _NC + lax.axis_index("c")
    table_cp = pltpu.async_copy(emb_hbm, table_v, sem)
    pltpu.sync_copy(x_hbm.at[pl.ds(w * BPW, BPW)], idx_v)
    table_cp.wait()

    def body(b, carry):
        accs = [jnp.zeros((_L,), jnp.float32) for _ in range(4)]
        for off, js in _TGROUPS:
            toks = idx_v[b, pl.ds(off, _L)]
            for j in js:
                row = toks[j]
                for k in range(2):
                    v = plsc.bitcast(
                        table_v[row, pl.ds(2 * k * _L, 2 * _L)], jnp.int32)
                    lo = lax.bitcast_convert_type(v << 16, jnp.float32)
                    hi = lax.bitcast_convert_type(v, jnp.float32)
                    accs[2 * k] = accs[2 * k] + lo
                    accs[2 * k + 1] = accs[2 * k + 1] + hi
        for k in range(4):
            pool_v[b, pl.ds(k * _L, _L)] = accs[k]
        return carry

    lax.fori_loop(0, BPW, body, jnp.int32(0))
    pltpu.sync_copy(pool_v, out_hbm.at[pl.ds(w * BPW, BPW)])


def _head_body(p_ref, wt_ref, b_ref, o_ref):
    logits = jnp.dot(p_ref[...], wt_ref[...],
                     preferred_element_type=jnp.float32)
    logits = logits + b_ref[...]
    m = jnp.max(logits, axis=-1, keepdims=True)
    e = jnp.exp(logits - m)
    o_ref[...] = e * (1.0 / jnp.sum(e, axis=-1, keepdims=True))


_HEAD_BLOCK = 512
_head = pl.pallas_call(
    _head_body,
    grid=(B // _HEAD_BLOCK,),
    in_specs=[
        pl.BlockSpec((_HEAD_BLOCK, DIM), lambda i: (i, 0)),
        pl.BlockSpec((DIM, OUT), lambda i: (0, 0)),
        pl.BlockSpec((1, OUT), lambda i: (0, 0)),
    ],
    out_specs=pl.BlockSpec((_HEAD_BLOCK, OUT), lambda i: (i, 0)),
    out_shape=jax.ShapeDtypeStruct((B, OUT), jnp.float32),
)

_PERM = np.array(
    [32 * (p // 32) + 2 * (p % 16) + ((p // 16) % 2) for p in range(DIM)],
    dtype=np.int32,
)


def kernel(x, emb, W, b):
    pooled_sum = _sc_pool(emb.astype(jnp.bfloat16), x.astype(jnp.int32))
    wt = W.T.astype(jnp.float32)[_PERM] * (1.0 / SENT)
    return _head(pooled_sum, wt, b.reshape(1, OUT))
